# baseline (speedup 1.0000x reference)
import functools

import jax
import jax.numpy as jnp
import numpy as np
from jax import lax
from jax.experimental import pallas as pl
from jax.experimental.pallas import tpu as pltpu
from jax.experimental.pallas import tpu_sc as plsc

N_SYM = 10000
N_TERM = 10000
E = 160000
D = 256
NW = 32
OWN = 320
N_PAD = NW * OWN
ACC_ROWS = OWN + 8
CNT_ROWS = OWN + 24
S = 1600
N_CHUNKS = E // S
G = 64
GS = G + 32

_PERM = np.arange(D).reshape(D // 32, 16, 2).transpose(0, 2, 1).reshape(D)


def _sc_segment_sums(term_x, src_idx, dst_idx):
    mesh = plsc.VectorSubcoreMesh(core_axis_name="c", subcore_axis_name="s")

    @functools.partial(
        pl.kernel,
        out_type=(jax.ShapeDtypeStruct((N_PAD, D), jnp.float32),
                  jax.ShapeDtypeStruct((N_PAD,), jnp.float32)),
        mesh=mesh,
        compiler_params=pltpu.CompilerParams(needs_layout_passes=False),
        scratch_types=[
            pltpu.VMEM((2 * S,), jnp.int32),
            pltpu.VMEM((2 * S,), jnp.int32),
            pltpu.VMEM((S + 160,), jnp.int32),
            pltpu.VMEM((S + 160,), jnp.int32),
            pltpu.VMEM((2 * GS,), jnp.int32),
            pltpu.VMEM((2 * GS,), jnp.int32),
            pltpu.VMEM((2 * G, D // 2), jnp.int32),
            pltpu.VMEM((ACC_ROWS, D), jnp.float32),
            pltpu.VMEM((CNT_ROWS,), jnp.float32),
            pltpu.SemaphoreType.DMA,
            pltpu.SemaphoreType.DMA,
            pltpu.SemaphoreType.DMA,
            pltpu.SemaphoreType.DMA,
        ],
    )
    def seg_kernel(term_hbm, src_hbm, dst_hbm, outs_hbm, outc_hbm, ssrc, sdst,
                   csrc, cldst, gidx, gld, rowbuf, accum, cnt,
                   esem1, esem2, gsem0, gsem1):
        cid = lax.axis_index("c")
        sid = lax.axis_index("s")
        base = (cid * 16 + sid) * OWN

        z16 = jnp.zeros((16,), jnp.float32)
        c16 = jnp.where(lax.iota(jnp.int32, 16) == 0, 1.0, 0.0)

        def zero_body(r, carry):
            for k in range(D // 16):
                accum[r, pl.ds(k * 16, 16)] = z16
            return carry

        lax.fori_loop(0, ACC_ROWS, zero_body, 0)
        for j in range(CNT_ROWS // 16 + 1):
            cnt[pl.ds(min(j * 16, CNT_ROWS - 16), 16)] = z16

        def issue_block(off, par):
            for j in range(G // 16):
                gidx[pl.ds(par * GS + j * 16, 16)] = (
                    csrc[pl.ds(off + j * 16, 16)])
                gld[pl.ds(par * GS + j * 16, 16)] = (
                    cldst[pl.ds(off + j * 16, 16)])

            @pl.when(par == 0)
            def _():
                pltpu.async_copy(term_hbm.at[gidx.at[pl.ds(0, G)]],
                                 rowbuf.at[pl.ds(0, G)], gsem0)

            @pl.when(par == 1)
            def _():
                pltpu.async_copy(term_hbm.at[gidx.at[pl.ds(GS, G)]],
                                 rowbuf.at[pl.ds(G, G)], gsem1)

        def wait_rmw(par):
            @pl.when(par == 0)
            def _():
                pltpu.make_async_copy(term_hbm.at[gidx.at[pl.ds(0, G)]],
                                      rowbuf.at[pl.ds(0, G)], gsem0).wait()

            @pl.when(par == 1)
            def _():
                pltpu.make_async_copy(term_hbm.at[gidx.at[pl.ds(GS, G)]],
                                      rowbuf.at[pl.ds(G, G)], gsem1).wait()

            def one_edge(e):
                r = gld[pl.ds(par * GS + e, 16)][0]
                for k in range(D // 32):
                    w16 = rowbuf[par * G + e, pl.ds(k * 16, 16)]
                    b32 = plsc.bitcast(w16, jnp.bfloat16)
                    lo, hi = plsc.unpack(b32, format=plsc.PackFormat.INTERLEAVED)
                    accum[r, pl.ds(k * 32, 16)] = (
                        accum[r, pl.ds(k * 32, 16)]
                        + lo.astype(jnp.float32))
                    accum[r, pl.ds(k * 32 + 16, 16)] = (
                        accum[r, pl.ds(k * 32 + 16, 16)]
                        + hi.astype(jnp.float32))
                cnt[pl.ds(r, 16)] = cnt[pl.ds(r, 16)] + c16

            def rmw(t, carry):
                one_edge(2 * t)
                one_edge(2 * t + 1)
                return carry

            lax.fori_loop(0, G // 2, rmw, 0)

        pltpu.async_copy(src_hbm.at[pl.ds(0, S)], ssrc.at[pl.ds(0, S)], esem1)
        pltpu.async_copy(dst_hbm.at[pl.ds(0, S)], sdst.at[pl.ds(0, S)], esem2)

        def chunk_body(c, st):
            nvec, pend, par = st
            cp = lax.rem(c, 2)
            pltpu.make_async_copy(src_hbm.at[pl.ds(c * S, S)],
                                  ssrc.at[pl.ds(cp * S, S)], esem1).wait()
            pltpu.make_async_copy(dst_hbm.at[pl.ds(c * S, S)],
                                  sdst.at[pl.ds(cp * S, S)], esem2).wait()

            @pl.when(c + 1 < N_CHUNKS)
            def _():
                np_ = 1 - cp
                pltpu.async_copy(src_hbm.at[pl.ds((c + 1) * S, S)],
                                 ssrc.at[pl.ds(np_ * S, S)], esem1)
                pltpu.async_copy(dst_hbm.at[pl.ds((c + 1) * S, S)],
                                 sdst.at[pl.ds(np_ * S, S)], esem2)

            for g in range(S // 16):
                ld = sdst[pl.ds(cp * S + g * 16, 16)] - base
                m = ld.astype(jnp.uint32) < jnp.uint32(OWN)
                nvec_in = nvec
                nvec = nvec + plsc.all_reduce_population_count(m)

                @pl.when(jnp.any(m))
                def _(g=g, ld=ld, m=m, nvec_in=nvec_in):
                    s16 = ssrc[pl.ds(cp * S + g * 16, 16)]
                    pos = plsc.cumsum(jnp.where(m, 1, 0))
                    idx = nvec_in + pos - 1
                    plsc.store_scatter(csrc, [idx], s16, mask=m)
                    plsc.store_scatter(cldst, [idx], ld, mask=m)

            n = nvec[0]
            nd = lax.div(n, G)

            def drain(t, st2):
                pend2, par2 = st2
                issue_block(t * G, par2)

                @pl.when(pend2 == 1)
                def _():
                    wait_rmw(1 - par2)

                return (1, 1 - par2)

            pend, par = lax.fori_loop(0, nd, drain, (pend, par))

            @pl.when(nd > 0)
            def _():
                for j in range(G // 16):
                    csrc[pl.ds(j * 16, 16)] = csrc[pl.ds(nd * G + j * 16, 16)]
                    cldst[pl.ds(j * 16, 16)] = (
                        cldst[pl.ds(nd * G + j * 16, 16)])

            return (jnp.broadcast_to(n - nd * G, (16,)), pend, par)

        nvec, pend, par = lax.fori_loop(
            0, N_CHUNKS, chunk_body,
            (jnp.zeros((16,), jnp.int32), jnp.int32(0), jnp.int32(0)))
        n = nvec[0]

        zi16 = jnp.zeros((16,), jnp.int32)
        di16 = jnp.full((16,), OWN, jnp.int32)
        for j in range(G // 16):
            csrc[pl.ds(n + j * 16, 16)] = zi16
            cldst[pl.ds(n + j * 16, 16)] = di16
        issue_block(0, par)

        @pl.when(pend == 1)
        def _():
            wait_rmw(1 - par)

        wait_rmw(par)

        pltpu.sync_copy(accum.at[pl.ds(0, OWN)],
                        outs_hbm.at[pl.ds(base, OWN)])
        pltpu.sync_copy(cnt.at[pl.ds(0, OWN)],
                        outc_hbm.at[pl.ds(base, OWN)])

    return seg_kernel(term_x, src_idx, dst_idx)


def _tc_finish_body(part_ref, cnt_ref, sym_ref, raw_ref, tt_ref, cw_ref,
                    cb_ref, wl_ref, bl_ref, wr_ref, g_ref, b_ref, out_ref):
    arity = raw_ref[:, 1:2]
    sraw = raw_ref[:, 2:3]
    tid = jnp.where(arity == 0, 2, jnp.where(sraw > 0, 0, 1))
    tt = tt_ref[...]
    emb = jnp.where(tid == 0, tt[0:1, :],
                    jnp.where(tid == 1, tt[1:2, :], tt[2:3, :]))
    cat = jnp.concatenate([sym_ref[...], emb], axis=1)
    enr = jax.nn.relu(
        lax.dot_general(cat, cw_ref[...], (((1,), (1,)), ((), ())),
                        preferred_element_type=jnp.float32) + cb_ref[...])
    sums = part_ref[...]
    cnt = cnt_ref[...]
    mean = sums / jnp.maximum(cnt, 1.0)
    conv = (lax.dot_general(mean, wl_ref[...], (((1,), (1,)), ((), ())),
                            preferred_element_type=jnp.float32)
            + bl_ref[...]
            + lax.dot_general(enr, wr_ref[...], (((1,), (1,)), ((), ())),
                              preferred_element_type=jnp.float32))
    h = conv + enr
    m = jnp.mean(h, axis=1, keepdims=True)
    v = jnp.mean((h - m) ** 2, axis=1, keepdims=True)
    out_ref[...] = (h - m) * lax.rsqrt(v + 1e-5) * g_ref[...] + b_ref[...]


def _tc_finish(partial, cnt, sym_x, sym_raw, type_table, combine_W, combine_b,
               sage_Wl, sage_bl, sage_Wr, ln_gamma, ln_beta):
    R = 1000
    grid = (N_SYM // R,)
    full = lambda i: (0, 0)
    row_blk = lambda i: (i, 0)
    return pl.pallas_call(
        _tc_finish_body,
        grid=grid,
        in_specs=[
            pl.BlockSpec((R, D), row_blk),
            pl.BlockSpec((R, 1), row_blk),
            pl.BlockSpec((R, D), row_blk),
            pl.BlockSpec((R, 3), row_blk),
            pl.BlockSpec((3, D), full),
            pl.BlockSpec((D, 2 * D), full),
            pl.BlockSpec((1, D), full),
            pl.BlockSpec((D, D), full),
            pl.BlockSpec((1, D), full),
            pl.BlockSpec((D, D), full),
            pl.BlockSpec((1, D), full),
            pl.BlockSpec((1, D), full),
        ],
        out_specs=pl.BlockSpec((R, D), row_blk),
        out_shape=jax.ShapeDtypeStruct((N_SYM, D), jnp.float32),
    )(partial, cnt, sym_x, sym_raw, type_table, combine_W,
      combine_b.reshape(1, D), sage_Wl, sage_bl.reshape(1, D), sage_Wr,
      ln_gamma.reshape(1, D), ln_beta.reshape(1, D))


def kernel(sym_x, term_x, type_table, combine_W, combine_b, sage_Wl, sage_bl,
           sage_Wr, ln_gamma, ln_beta, sym_raw, edge_index):
    term_i32 = lax.bitcast_convert_type(
        term_x.astype(jnp.bfloat16).reshape(N_TERM, D // 2, 2), jnp.int32)
    partial, cnt = _sc_segment_sums(term_i32, edge_index[0], edge_index[1])
    return _tc_finish(partial, cnt.reshape(N_PAD, 1), sym_x, sym_raw,
                      type_table, combine_W, combine_b, sage_Wl[:, _PERM],
                      sage_bl, sage_Wr, ln_gamma, ln_beta)


# speedup vs baseline: 1.3235x; 1.3235x over previous
import functools

import jax
import jax.numpy as jnp
import numpy as np
from jax import lax
from jax.experimental import pallas as pl
from jax.experimental.pallas import tpu as pltpu
from jax.experimental.pallas import tpu_sc as plsc

N_SYM = 10000
N_TERM = 10000
E = 160000
D = 256
NW = 32
OWN = 320
N_PAD = NW * OWN
ACC_ROWS = OWN + 8
CNT_ROWS = OWN + 24
S = 1600
N_CHUNKS = E // S
G = 64
GS = G + 32

_PERM = np.arange(D).reshape(D // 32, 16, 2).transpose(0, 2, 1).reshape(D)


def _sc_segment_sums(term_x, src_idx, dst_idx):
    mesh = plsc.VectorSubcoreMesh(core_axis_name="c", subcore_axis_name="s")

    @functools.partial(
        pl.kernel,
        out_type=(jax.ShapeDtypeStruct((N_PAD, D), jnp.float32),
                  jax.ShapeDtypeStruct((N_PAD,), jnp.float32)),
        mesh=mesh,
        compiler_params=pltpu.CompilerParams(needs_layout_passes=False),
        scratch_types=[
            pltpu.VMEM((2 * S,), jnp.int32),
            pltpu.VMEM((2 * S,), jnp.int32),
            pltpu.VMEM((S + 160,), jnp.int32),
            pltpu.VMEM((S + 160,), jnp.int32),
            pltpu.VMEM((2 * GS,), jnp.int32),
            pltpu.VMEM((2 * GS,), jnp.int32),
            pltpu.VMEM((2 * G, D // 2), jnp.int32),
            pltpu.VMEM((ACC_ROWS, D), jnp.float32),
            pltpu.VMEM((CNT_ROWS,), jnp.float32),
            pltpu.SemaphoreType.DMA,
            pltpu.SemaphoreType.DMA,
            pltpu.SemaphoreType.DMA,
            pltpu.SemaphoreType.DMA,
        ],
    )
    def seg_kernel(term_hbm, src_hbm, dst_hbm, outs_hbm, outc_hbm, ssrc, sdst,
                   csrc, cldst, gidx, gld, rowbuf, accum, cnt,
                   esem1, esem2, gsem0, gsem1):
        cid = lax.axis_index("c")
        sid = lax.axis_index("s")
        base = (cid * 16 + sid) * OWN

        z16 = jnp.zeros((16,), jnp.float32)
        c16 = jnp.where(lax.iota(jnp.int32, 16) == 0, 1.0, 0.0)

        def zero_body(r, carry):
            for k in range(D // 16):
                accum[r, pl.ds(k * 16, 16)] = z16
            return carry

        lax.fori_loop(0, ACC_ROWS, zero_body, 0)
        for j in range(CNT_ROWS // 16 + 1):
            cnt[pl.ds(min(j * 16, CNT_ROWS - 16), 16)] = z16

        def issue_block(off, par):
            for j in range(G // 16):
                gidx[pl.ds(par * GS + j * 16, 16)] = (
                    csrc[pl.ds(off + j * 16, 16)])
                gld[pl.ds(par * GS + j * 16, 16)] = (
                    cldst[pl.ds(off + j * 16, 16)])

            @pl.when(par == 0)
            def _():
                pltpu.async_copy(term_hbm.at[gidx.at[pl.ds(0, G)]],
                                 rowbuf.at[pl.ds(0, G)], gsem0)

            @pl.when(par == 1)
            def _():
                pltpu.async_copy(term_hbm.at[gidx.at[pl.ds(GS, G)]],
                                 rowbuf.at[pl.ds(G, G)], gsem1)

        def wait_rmw(par):
            @pl.when(par == 0)
            def _():
                pltpu.make_async_copy(term_hbm.at[gidx.at[pl.ds(0, G)]],
                                      rowbuf.at[pl.ds(0, G)], gsem0).wait()

            @pl.when(par == 1)
            def _():
                pltpu.make_async_copy(term_hbm.at[gidx.at[pl.ds(GS, G)]],
                                      rowbuf.at[pl.ds(G, G)], gsem1).wait()

            def one_edge(e):
                r = gld[pl.ds(par * GS + e, 16)][0]
                for k in range(D // 32):
                    w16 = rowbuf[par * G + e, pl.ds(k * 16, 16)]
                    b32 = plsc.bitcast(w16, jnp.bfloat16)
                    lo, hi = plsc.unpack(b32, format=plsc.PackFormat.INTERLEAVED)
                    accum[r, pl.ds(k * 32, 16)] = (
                        accum[r, pl.ds(k * 32, 16)]
                        + lo.astype(jnp.float32))
                    accum[r, pl.ds(k * 32 + 16, 16)] = (
                        accum[r, pl.ds(k * 32 + 16, 16)]
                        + hi.astype(jnp.float32))
                cnt[pl.ds(r, 16)] = cnt[pl.ds(r, 16)] + c16

            def rmw(t, carry):
                one_edge(2 * t)
                one_edge(2 * t + 1)
                return carry

            lax.fori_loop(0, G // 2, rmw, 0)

        pltpu.async_copy(src_hbm.at[pl.ds(0, S)], ssrc.at[pl.ds(0, S)], esem1)
        pltpu.async_copy(dst_hbm.at[pl.ds(0, S)], sdst.at[pl.ds(0, S)], esem2)

        def chunk_body(c, st):
            nvec, pend, par = st
            cp = lax.rem(c, 2)
            pltpu.make_async_copy(src_hbm.at[pl.ds(c * S, S)],
                                  ssrc.at[pl.ds(cp * S, S)], esem1).wait()
            pltpu.make_async_copy(dst_hbm.at[pl.ds(c * S, S)],
                                  sdst.at[pl.ds(cp * S, S)], esem2).wait()

            @pl.when(c + 1 < N_CHUNKS)
            def _():
                np_ = 1 - cp
                pltpu.async_copy(src_hbm.at[pl.ds((c + 1) * S, S)],
                                 ssrc.at[pl.ds(np_ * S, S)], esem1)
                pltpu.async_copy(dst_hbm.at[pl.ds((c + 1) * S, S)],
                                 sdst.at[pl.ds(np_ * S, S)], esem2)

            for g in range(S // 16):
                s16 = ssrc[pl.ds(cp * S + g * 16, 16)]
                ld = sdst[pl.ds(cp * S + g * 16, 16)] - base
                m = ld.astype(jnp.uint32) < jnp.uint32(OWN)
                pos = plsc.cumsum(jnp.where(m, 1, 0))
                idx = nvec + pos - 1
                plsc.store_scatter(csrc, [idx], s16, mask=m)
                plsc.store_scatter(cldst, [idx], ld, mask=m)
                nvec = nvec + plsc.all_reduce_population_count(m)

            n = nvec[0]
            nd = lax.div(n, G)

            def drain(t, st2):
                pend2, par2 = st2
                issue_block(t * G, par2)

                @pl.when(pend2 == 1)
                def _():
                    wait_rmw(1 - par2)

                return (1, 1 - par2)

            pend, par = lax.fori_loop(0, nd, drain, (pend, par))

            @pl.when(nd > 0)
            def _():
                for j in range(G // 16):
                    csrc[pl.ds(j * 16, 16)] = csrc[pl.ds(nd * G + j * 16, 16)]
                    cldst[pl.ds(j * 16, 16)] = (
                        cldst[pl.ds(nd * G + j * 16, 16)])

            return (jnp.broadcast_to(n - nd * G, (16,)), pend, par)

        nvec, pend, par = lax.fori_loop(
            0, N_CHUNKS, chunk_body,
            (jnp.zeros((16,), jnp.int32), jnp.int32(0), jnp.int32(0)))
        n = nvec[0]

        zi16 = jnp.zeros((16,), jnp.int32)
        di16 = jnp.full((16,), OWN, jnp.int32)
        for j in range(G // 16):
            csrc[pl.ds(n + j * 16, 16)] = zi16
            cldst[pl.ds(n + j * 16, 16)] = di16
        issue_block(0, par)

        @pl.when(pend == 1)
        def _():
            wait_rmw(1 - par)

        wait_rmw(par)

        pltpu.sync_copy(accum.at[pl.ds(0, OWN)],
                        outs_hbm.at[pl.ds(base, OWN)])
        pltpu.sync_copy(cnt.at[pl.ds(0, OWN)],
                        outc_hbm.at[pl.ds(base, OWN)])

    return seg_kernel(term_x, src_idx, dst_idx)


def _tc_finish_body(part_ref, cnt_ref, sym_ref, raw_ref, tt_ref, cw_ref,
                    cb_ref, wl_ref, bl_ref, wr_ref, g_ref, b_ref, out_ref):
    arity = raw_ref[:, 1:2]
    sraw = raw_ref[:, 2:3]
    tid = jnp.where(arity == 0, 2, jnp.where(sraw > 0, 0, 1))
    tt = tt_ref[...]
    emb = jnp.where(tid == 0, tt[0:1, :],
                    jnp.where(tid == 1, tt[1:2, :], tt[2:3, :]))
    cat = jnp.concatenate([sym_ref[...], emb], axis=1)
    enr = jax.nn.relu(
        lax.dot_general(cat, cw_ref[...], (((1,), (1,)), ((), ())),
                        preferred_element_type=jnp.float32) + cb_ref[...])
    sums = part_ref[...]
    cnt = cnt_ref[...]
    mean = sums / jnp.maximum(cnt, 1.0)
    conv = (lax.dot_general(mean, wl_ref[...], (((1,), (1,)), ((), ())),
                            preferred_element_type=jnp.float32)
            + bl_ref[...]
            + lax.dot_general(enr, wr_ref[...], (((1,), (1,)), ((), ())),
                              preferred_element_type=jnp.float32))
    h = conv + enr
    m = jnp.mean(h, axis=1, keepdims=True)
    v = jnp.mean((h - m) ** 2, axis=1, keepdims=True)
    out_ref[...] = (h - m) * lax.rsqrt(v + 1e-5) * g_ref[...] + b_ref[...]


def _tc_finish(partial, cnt, sym_x, sym_raw, type_table, combine_W, combine_b,
               sage_Wl, sage_bl, sage_Wr, ln_gamma, ln_beta):
    R = 1000
    grid = (N_SYM // R,)
    full = lambda i: (0, 0)
    row_blk = lambda i: (i, 0)
    return pl.pallas_call(
        _tc_finish_body,
        grid=grid,
        in_specs=[
            pl.BlockSpec((R, D), row_blk),
            pl.BlockSpec((R, 1), row_blk),
            pl.BlockSpec((R, D), row_blk),
            pl.BlockSpec((R, 3), row_blk),
            pl.BlockSpec((3, D), full),
            pl.BlockSpec((D, 2 * D), full),
            pl.BlockSpec((1, D), full),
            pl.BlockSpec((D, D), full),
            pl.BlockSpec((1, D), full),
            pl.BlockSpec((D, D), full),
            pl.BlockSpec((1, D), full),
            pl.BlockSpec((1, D), full),
        ],
        out_specs=pl.BlockSpec((R, D), row_blk),
        out_shape=jax.ShapeDtypeStruct((N_SYM, D), jnp.float32),
    )(partial, cnt, sym_x, sym_raw, type_table, combine_W,
      combine_b.reshape(1, D), sage_Wl, sage_bl.reshape(1, D), sage_Wr,
      ln_gamma.reshape(1, D), ln_beta.reshape(1, D))


def kernel(sym_x, term_x, type_table, combine_W, combine_b, sage_Wl, sage_bl,
           sage_Wr, ln_gamma, ln_beta, sym_raw, edge_index):
    term_i32 = lax.bitcast_convert_type(
        term_x.astype(jnp.bfloat16).reshape(N_TERM, D // 2, 2), jnp.int32)
    partial, cnt = _sc_segment_sums(term_i32, edge_index[0], edge_index[1])
    return _tc_finish(partial, cnt.reshape(N_PAD, 1), sym_x, sym_raw,
                      type_table, combine_W, combine_b, sage_Wl[:, _PERM],
                      sage_bl, sage_Wr, ln_gamma, ln_beta)
